# Initial kernel scaffold; baseline (speedup 1.0000x reference)
import jax
import jax.numpy as jnp
from jax.experimental import pallas as pl


def kernel(logits, top_p, top_k, min_p, soft_mask):
    raise NotImplementedError("write your pallas kernel here")


# speedup vs baseline: 1.0063x; 1.0063x over previous
import functools

import jax
import jax.numpy as jnp
from jax import lax
from jax.experimental import pallas as pl

_B = 128
_VOCAB = 100000
_K = 2048


def _sample_body(ps_ref, pidx_ref, tp_ref, tk_ref, mp_ref, sm_ref, g_ref,
                 outp_ref, outi_ref, tok_ref):
    ps = ps_ref[...]
    pidx = pidx_ref[...]
    tp = tp_ref[...]
    tk = tk_ref[...]
    mp = mp_ref[...]
    sm = sm_ref[...]
    g = g_ref[...]

    offs = lax.broadcasted_iota(jnp.int32, (_B, _K), 1)
    cs = ps
    d = 1
    while d < _K:
        shifted = jnp.concatenate(
            [jnp.zeros((_B, d), jnp.float32), cs[:, :_K - d]], axis=1)
        cs = cs + shifted
        d *= 2
    mask_k = offs < tk
    mask_p = jnp.logical_not(cs - ps > tp)
    apply_minp = mp > 0.0
    thr = jnp.where(apply_minp & (tk > 0), ps[:, 0:1] * mp, 0.0)
    minp_out = apply_minp & (ps < thr)
    final_mask = mask_k & mask_p & jnp.logical_not(minp_out)
    filtered = jnp.where(final_mask, ps, 0.0)
    denom = jnp.sum(filtered, axis=-1, keepdims=True)
    denom_safe = jnp.where(denom == 0.0, 1.0, denom)
    normed = filtered / denom_safe
    normed = jnp.where((denom == 0.0) & (offs == 0), 1.0, normed)
    logp = jnp.where(normed > 0.0, jnp.log(jnp.maximum(normed, 1e-38)),
                     -jnp.inf)
    score = logp + g
    smax = jnp.max(score, axis=-1, keepdims=True)
    jstar = jnp.min(jnp.where(score == smax, offs, _K), axis=-1, keepdims=True)
    sel = offs == jstar
    tok = jnp.sum(jnp.where(sel, pidx, 0), axis=-1, keepdims=True)
    std_probs = jnp.where(sel, 1.0, 0.0).astype(jnp.float32)
    std_idx = jnp.where(sel, tok, 0)
    is_soft = sm > 0
    outp_ref[...] = jnp.where(is_soft, normed, std_probs)
    outi_ref[...] = jnp.where(is_soft, pidx, std_idx)
    tok_ref[...] = tok


@functools.partial(jax.jit, static_argnames=("interpret",))
def _run(logits, top_p, top_k, min_p, soft_mask, interpret=False):
    top_k = jnp.maximum(top_k, 1)
    probs = jax.nn.softmax(logits.astype(jnp.float32), axis=-1)
    probs_sort, probs_idx = jax.lax.top_k(probs, _K)
    noise = jax.random.gumbel(jax.random.key(1234), (_B, _K), jnp.float32)

    outp, outi, tok = pl.pallas_call(
        _sample_body,
        out_shape=(
            jax.ShapeDtypeStruct((_B, _K), jnp.float32),
            jax.ShapeDtypeStruct((_B, _K), jnp.int32),
            jax.ShapeDtypeStruct((_B, 1), jnp.int32),
        ),
        interpret=interpret,
    )(probs_sort, probs_idx,
      top_p[:, None], top_k[:, None], min_p[:, None],
      soft_mask[:, None].astype(jnp.int32), noise)
    return outp, outi, tok[:, 0]


def kernel(logits, top_p, top_k, min_p, soft_mask):
    return _run(logits, top_p, top_k, min_p, soft_mask)


# speedup vs baseline: 6.3125x; 6.2731x over previous
import functools

import jax
import jax.numpy as jnp
import numpy as np
from jax import lax
from jax.experimental import pallas as pl
from jax.experimental.pallas import tpu as pltpu, tpu_sc as plsc

_B = 128
_V = 100000
_K = 2048
_NW = 32
_RPW = _B // _NW
_BINS = 8192
_SHIFT = 19
_CCAP = 4096
_VREGS = _V // 16
_HV = _BINS // 16

_TOPBIT = np.uint32(0x80000000)
_ALLONES = np.uint32(0xFFFFFFFF)


def _sortable_u32(x):
    b = plsc.bitcast(x, jnp.uint32)
    neg = b >> 31
    return b ^ ((np.uint32(0) - neg) | _TOPBIT)


def _unsortable_f32(u):
    pos = (u ^ _ALLONES) >> 31
    return plsc.bitcast(u ^ ((np.uint32(0) - pos) | _TOPBIT), jnp.float32)


def _sc_topk(logits_flat):
    mesh = plsc.VectorSubcoreMesh(core_axis_name="c", subcore_axis_name="s")

    @functools.partial(
        pl.kernel,
        out_type=(
            jax.ShapeDtypeStruct((_B * _K,), jnp.float32),
            jax.ShapeDtypeStruct((_B * _K,), jnp.int32),
            jax.ShapeDtypeStruct((_B * 16,), jnp.float32),
        ),
        mesh=mesh,
        compiler_params=pltpu.CompilerParams(needs_layout_passes=False),
        scratch_types=[
            pltpu.VMEM((_V,), jnp.float32),
            pltpu.VMEM((_BINS,), jnp.int32),
            pltpu.VMEM((_CCAP + 32,), jnp.int32),
            pltpu.VMEM((_CCAP + 32,), jnp.int32),
            pltpu.VMEM((_CCAP + 32,), jnp.int32),
            pltpu.VMEM((_CCAP + 32,), jnp.int32),
            pltpu.VMEM((256,), jnp.int32),
            pltpu.VMEM((_K,), jnp.float32),
            pltpu.VMEM((16,), jnp.float32),
            pltpu.SemaphoreType.DMA,
        ],
    )
    def topk(logits_hbm, vals_hbm, ids_hbm, mz_hbm,
             row, hist, ka, ia, kb, ib, offs, stage, mzst, sem):
        wid = lax.axis_index("s") * 2 + lax.axis_index("c")
        lane = lax.iota(jnp.int32, 16)
        zero16 = jnp.zeros((16,), jnp.int32)

        for r in range(_RPW):
            rix = wid * _RPW + r
            pltpu.async_copy(
                logits_hbm.at[pl.ds(rix * _V, _V)], row, sem).wait()

            def zh_body(j, c):
                hist[pl.ds(j * 16, 16)] = zero16
                return c
            lax.fori_loop(0, _HV, zh_body, 0)

            def p1_body(i, maxv):
                x = row[pl.ds(i * 16, 16)]
                u = _sortable_u32(x)
                b = plsc.bitcast(u >> _SHIFT, jnp.int32)
                occ, last = plsc.scan_count(b)
                plsc.addupdate_scatter(hist, [b], occ, mask=last)
                return jnp.maximum(maxv, x)
            maxv = lax.fori_loop(0, _VREGS, p1_body,
                                 jnp.full((16,), -jnp.inf, jnp.float32))
            m = jnp.max(maxv)

            def p2_body(j, carry):
                fprev, cnt = carry
                h = hist[pl.ds(j * 16, 16)]
                cs = plsc.cumsum(h) + fprev
                cnt = cnt + jnp.sum((cs <= (_V - _K)).astype(jnp.int32))
                return fprev + jnp.sum(h), cnt
            _, bstar = lax.fori_loop(0, _HV, p2_body, (0, 0))
            ts = (bstar << _SHIFT) ^ np.int32(-2147483648)

            def p3_body(i, carry):
                zsum, cnt = carry
                x = row[pl.ds(i * 16, 16)]
                zsum = zsum + jnp.exp(x - m)
                u = _sortable_u32(x)
                s = plsc.bitcast(u ^ _TOPBIT, jnp.int32)
                keep = (s >= ts) & (jnp.full((16,), cnt, jnp.int32)
                                    <= (_CCAP - 16))
                d = plsc.bitcast(u ^ _ALLONES, jnp.int32)
                gid = jnp.full((16,), i * 16, jnp.int32) + lane
                plsc.store_compressed(ka.at[pl.ds(cnt, 16)], d, mask=keep)
                plsc.store_compressed(ia.at[pl.ds(cnt, 16)], gid, mask=keep)
                return zsum, cnt + jnp.sum(keep.astype(jnp.int32))
            zsum, cnt = lax.fori_loop(
                0, _VREGS, p3_body, (jnp.zeros((16,), jnp.float32), 0))
            zval = jnp.sum(zsum)
            ka[pl.ds(cnt, 16)] = jnp.full((16,), 0x7FFFFFFF, jnp.int32)
            ia[pl.ds(cnt, 16)] = zero16
            nv = (cnt + 15) >> 4

            for p, (src_k, src_i, dst_k, dst_i) in enumerate(
                    ((ka, ia, kb, ib), (kb, ib, ka, ia),
                     (ka, ia, kb, ib), (kb, ib, ka, ia))):
                sh = np.uint32(8 * p)

                def zo_body(j, c):
                    offs[pl.ds(j * 16, 16)] = zero16
                    return c
                lax.fori_loop(0, 16, zo_body, 0)

                def rh_body(i, c):
                    d = plsc.bitcast(src_k[pl.ds(i * 16, 16)], jnp.uint32)
                    dig = plsc.bitcast((d >> sh) & np.uint32(255), jnp.int32)
                    occ, last = plsc.scan_count(dig)
                    plsc.addupdate_scatter(offs, [dig], occ, mask=last)
                    return c
                lax.fori_loop(0, nv, rh_body, 0)

                def rp_body(j, acc):
                    h = offs[pl.ds(j * 16, 16)]
                    cs = plsc.cumsum(h) + acc
                    offs[pl.ds(j * 16, 16)] = cs - h
                    return acc + jnp.sum(h)
                lax.fori_loop(0, 16, rp_body, 0)

                def rs_body(i, c):
                    dk = src_k[pl.ds(i * 16, 16)]
                    di = src_i[pl.ds(i * 16, 16)]
                    d = plsc.bitcast(dk, jnp.uint32)
                    dig = plsc.bitcast((d >> sh) & np.uint32(255), jnp.int32)
                    occ, last = plsc.scan_count(dig)
                    base = plsc.load_gather(offs, [dig])
                    slot = base + occ - 1
                    plsc.store_scatter(dst_k, [slot], dk)
                    plsc.store_scatter(dst_i, [slot], di)
                    plsc.addupdate_scatter(offs, [dig], occ, mask=last)
                    return c
                lax.fori_loop(0, nv, rs_body, 0)

            def o_body(i, c):
                d = plsc.bitcast(ka[pl.ds(i * 16, 16)], jnp.uint32)
                stage[pl.ds(i * 16, 16)] = _unsortable_f32(
                    d ^ _ALLONES)
                return c
            lax.fori_loop(0, _K // 16, o_body, 0)
            mzst[...] = jnp.where(
                lane == 0, m, jnp.where(lane == 1, zval, 0.0))
            pltpu.sync_copy(stage, vals_hbm.at[pl.ds(rix * _K, _K)])
            pltpu.sync_copy(ia.at[pl.ds(0, _K)], ids_hbm.at[pl.ds(rix * _K, _K)])
            pltpu.sync_copy(mzst, mz_hbm.at[pl.ds(rix * 16, 16)])

    return topk(logits_flat)


def _sample_body(ls_ref, pidx_ref, mz_ref, tp_ref, tk_ref, mp_ref, sm_ref,
                 g_ref, outp_ref, outi_ref, tok_ref):
    ls = ls_ref[...]
    pidx = pidx_ref[...]
    mz = mz_ref[...]
    tp = tp_ref[...]
    tk = tk_ref[...]
    mp = mp_ref[...]
    sm = sm_ref[...]
    g = g_ref[...]

    m = mz[:, 0:1]
    z = mz[:, 1:2]
    ps = jnp.exp(ls - m) / z

    offs = lax.broadcasted_iota(jnp.int32, (_B, _K), 1)
    cs = ps
    d = 1
    while d < _K:
        shifted = jnp.concatenate(
            [jnp.zeros((_B, d), jnp.float32), cs[:, :_K - d]], axis=1)
        cs = cs + shifted
        d *= 2
    mask_k = offs < tk
    mask_p = jnp.logical_not(cs - ps > tp)
    apply_minp = mp > 0.0
    thr = jnp.where(apply_minp & (tk > 0), ps[:, 0:1] * mp, 0.0)
    minp_out = apply_minp & (ps < thr)
    final_mask = mask_k & mask_p & jnp.logical_not(minp_out)
    filtered = jnp.where(final_mask, ps, 0.0)
    denom = jnp.sum(filtered, axis=-1, keepdims=True)
    denom_safe = jnp.where(denom == 0.0, 1.0, denom)
    normed = filtered / denom_safe
    normed = jnp.where((denom == 0.0) & (offs == 0), 1.0, normed)
    logp = jnp.where(normed > 0.0, jnp.log(jnp.maximum(normed, 1e-38)),
                     -jnp.inf)
    score = logp + g
    smax = jnp.max(score, axis=-1, keepdims=True)
    jstar = jnp.min(jnp.where(score == smax, offs, _K), axis=-1, keepdims=True)
    sel = offs == jstar
    tok = jnp.sum(jnp.where(sel, pidx, 0), axis=-1, keepdims=True)
    std_probs = jnp.where(sel, 1.0, 0.0).astype(jnp.float32)
    std_idx = jnp.where(sel, tok, 0)
    is_soft = sm > 0
    outp_ref[...] = jnp.where(is_soft, normed, std_probs)
    outi_ref[...] = jnp.where(is_soft, pidx, std_idx)
    tok_ref[...] = tok


@jax.jit
def _run(logits, top_p, top_k, min_p, soft_mask):
    top_k = jnp.maximum(top_k, 1)
    vals, ids, mz = _sc_topk(logits.reshape(-1))
    vals = vals.reshape(_B, _K)
    ids = ids.reshape(_B, _K)
    mz = mz.reshape(_B, 16)
    noise = jax.random.gumbel(jax.random.key(1234), (_B, _K), jnp.float32)

    outp, outi, tok = pl.pallas_call(
        _sample_body,
        out_shape=(
            jax.ShapeDtypeStruct((_B, _K), jnp.float32),
            jax.ShapeDtypeStruct((_B, _K), jnp.int32),
            jax.ShapeDtypeStruct((_B, 1), jnp.int32),
        ),
    )(vals, ids, mz,
      top_p[:, None], top_k[:, None], min_p[:, None],
      soft_mask[:, None].astype(jnp.int32), noise)
    return outp, outi, tok[:, 0]


def kernel(logits, top_p, top_k, min_p, soft_mask):
    return _run(logits, top_p, top_k, min_p, soft_mask)


# speedup vs baseline: 7.9263x; 1.2557x over previous
import functools

import jax
import jax.numpy as jnp
import numpy as np
from jax import lax
from jax.experimental import pallas as pl
from jax.experimental.pallas import tpu as pltpu, tpu_sc as plsc

_B = 128
_V = 100000
_K = 2048
_NW = 32
_RPW = _B // _NW
_BINS = 8192
_SHIFT = 19
_CCAP = 4096
_VREGS = _V // 16
_HV = _BINS // 16
_U = 10

_TOPBIT = np.uint32(0x80000000)
_ALLONES = np.uint32(0xFFFFFFFF)


def _sortable_u32(x):
    b = plsc.bitcast(x, jnp.uint32)
    neg = b >> 31
    return b ^ ((np.uint32(0) - neg) | _TOPBIT)


def _sc_topk(logits_flat):
    mesh = plsc.VectorSubcoreMesh(core_axis_name="c", subcore_axis_name="s")

    @functools.partial(
        pl.kernel,
        out_type=(
            jax.ShapeDtypeStruct((_B * _K,), jnp.float32),
            jax.ShapeDtypeStruct((_B * _K,), jnp.int32),
        ),
        mesh=mesh,
        compiler_params=pltpu.CompilerParams(needs_layout_passes=False),
        scratch_types=[
            pltpu.VMEM((_V,), jnp.float32),
            pltpu.VMEM((_BINS,), jnp.int32),
            pltpu.VMEM((_CCAP + 32,), jnp.int32),
            pltpu.VMEM((_CCAP + 32,), jnp.int32),
            pltpu.VMEM((_CCAP + 32,), jnp.int32),
            pltpu.VMEM((_CCAP + 32,), jnp.int32),
            pltpu.VMEM((256,), jnp.int32),
            pltpu.VMEM((_K,), jnp.float32),
            pltpu.SemaphoreType.DMA,
        ],
    )
    def topk(logits_hbm, vals_hbm, ids_hbm,
             row, hist, ka, ia, kb, ib, offs, stage, sem):
        wid = lax.axis_index("s") * 2 + lax.axis_index("c")
        lane = lax.iota(jnp.int32, 16)
        zero16 = jnp.zeros((16,), jnp.int32)

        for r in range(_RPW):
            rix = wid * _RPW + r
            pltpu.async_copy(
                logits_hbm.at[pl.ds(rix * _V, _V)], row, sem).wait()

            def zh_body(j, c):
                for t in range(8):
                    hist[pl.ds(j * 128 + t * 16, 16)] = zero16
                return c
            lax.fori_loop(0, _HV // 8, zh_body, 0)

            def p1_body(i, c):
                base = i * (16 * _U)
                for t in range(_U):
                    x = row[pl.ds(base + t * 16, 16)]
                    u = _sortable_u32(x)
                    b = plsc.bitcast(u >> _SHIFT, jnp.int32)
                    occ, last = plsc.scan_count(b)
                    plsc.addupdate_scatter(hist, [b], occ, mask=last)
                return c
            lax.fori_loop(0, _VREGS // _U, p1_body, 0)

            def p2_body(j, carry):
                fprev, cnt = carry
                for t in range(8):
                    h = hist[pl.ds(j * 128 + t * 16, 16)]
                    cs = plsc.cumsum(h) + fprev
                    cnt = cnt + jnp.sum((cs <= (_V - _K)).astype(jnp.int32))
                    fprev = fprev + jnp.sum(h)
                return fprev, cnt
            _, bstar = lax.fori_loop(0, _HV // 8, p2_body, (0, 0))
            tu = plsc.bitcast(jnp.full((16,), bstar << _SHIFT, jnp.int32),
                              jnp.uint32)
            pos = (tu ^ _ALLONES) >> 31
            tfv = plsc.bitcast(tu ^ ((np.uint32(0) - pos) | _TOPBIT),
                               jnp.float32)
            tf = jnp.max(tfv)

            def p3_body(i, carry):
                cnt, gid = carry
                base = i * (16 * _U)
                for t in range(_U):
                    x = row[pl.ds(base + t * 16, 16)]
                    keep = x >= tf
                    plsc.store_compressed(ia.at[pl.ds(cnt, 16)], gid,
                                          mask=keep)
                    cnt = jnp.minimum(
                        cnt + jnp.sum(keep.astype(jnp.int32)), _CCAP)
                    gid = gid + 16
                return cnt, gid
            cnt, _ = lax.fori_loop(0, _VREGS // _U, p3_body, (0, lane))
            ia[pl.ds(cnt, 16)] = zero16
            nv = (cnt + 15) >> 4

            def p3b_body(i, c):
                idx = ia[pl.ds(i * 16, 16)]
                x = plsc.load_gather(row, [idx])
                u = _sortable_u32(x)
                ka[pl.ds(i * 16, 16)] = plsc.bitcast(u ^ _ALLONES, jnp.int32)
                return c
            lax.fori_loop(0, nv, p3b_body, 0)
            ka[pl.ds(cnt, 16)] = jnp.full((16,), -1, jnp.int32)

            for p, (src_k, src_i, dst_k, dst_i) in enumerate(
                    ((ka, ia, kb, ib), (kb, ib, ka, ia),
                     (ka, ia, kb, ib), (kb, ib, ka, ia))):
                sh = np.uint32(8 * p)

                def zo_body(j, c):
                    offs[pl.ds(j * 16, 16)] = zero16
                    return c
                lax.fori_loop(0, 16, zo_body, 0)

                def rh_body(i, c):
                    d = plsc.bitcast(src_k[pl.ds(i * 16, 16)], jnp.uint32)
                    dig = plsc.bitcast((d >> sh) & np.uint32(255), jnp.int32)
                    occ, last = plsc.scan_count(dig)
                    plsc.addupdate_scatter(offs, [dig], occ, mask=last)
                    return c
                lax.fori_loop(0, nv, rh_body, 0)

                def rp_body(j, acc):
                    h = offs[pl.ds(j * 16, 16)]
                    cs = plsc.cumsum(h) + acc
                    offs[pl.ds(j * 16, 16)] = cs - h
                    return acc + jnp.sum(h)
                lax.fori_loop(0, 16, rp_body, 0)

                def rs_body(i, c):
                    dk = src_k[pl.ds(i * 16, 16)]
                    di = src_i[pl.ds(i * 16, 16)]
                    d = plsc.bitcast(dk, jnp.uint32)
                    dig = plsc.bitcast((d >> sh) & np.uint32(255), jnp.int32)
                    occ, last = plsc.scan_count(dig)
                    base = plsc.load_gather(offs, [dig])
                    slot = base + occ - 1
                    plsc.store_scatter(dst_k, [slot], dk)
                    plsc.store_scatter(dst_i, [slot], di)
                    plsc.addupdate_scatter(offs, [dig], occ, mask=last)
                    return c
                lax.fori_loop(0, nv, rs_body, 0)

            def o_body(i, c):
                d = plsc.bitcast(ka[pl.ds(i * 16, 16)], jnp.uint32)
                u = d ^ _ALLONES
                pos2 = (u ^ _ALLONES) >> 31
                stage[pl.ds(i * 16, 16)] = plsc.bitcast(
                    u ^ ((np.uint32(0) - pos2) | _TOPBIT), jnp.float32)
                return c
            lax.fori_loop(0, _K // 16, o_body, 0)
            pltpu.sync_copy(stage, vals_hbm.at[pl.ds(rix * _K, _K)])
            pltpu.sync_copy(ia.at[pl.ds(0, _K)],
                            ids_hbm.at[pl.ds(rix * _K, _K)])

    return topk(logits_flat)


def _mz_body(x_ref, m_ref, z_ref):
    x = x_ref[...]
    m = jnp.max(x, axis=-1, keepdims=True)
    m_ref[...] = m
    z_ref[...] = jnp.sum(jnp.exp(x - m), axis=-1, keepdims=True)


def _sample_body(ls_ref, pidx_ref, m_ref, z_ref, tp_ref, tk_ref, mp_ref,
                 sm_ref, g_ref, outp_ref, outi_ref, tok_ref):
    ls = ls_ref[...]
    pidx = pidx_ref[...]
    m = m_ref[...]
    z = z_ref[...]
    tp = tp_ref[...]
    tk = tk_ref[...]
    mp = mp_ref[...]
    sm = sm_ref[...]
    g = g_ref[...]

    ps = jnp.exp(ls - m) / z

    offs = lax.broadcasted_iota(jnp.int32, (_B, _K), 1)
    cs = ps
    d = 1
    while d < _K:
        shifted = jnp.concatenate(
            [jnp.zeros((_B, d), jnp.float32), cs[:, :_K - d]], axis=1)
        cs = cs + shifted
        d *= 2
    mask_k = offs < tk
    mask_p = jnp.logical_not(cs - ps > tp)
    apply_minp = mp > 0.0
    thr = jnp.where(apply_minp & (tk > 0), ps[:, 0:1] * mp, 0.0)
    minp_out = apply_minp & (ps < thr)
    final_mask = mask_k & mask_p & jnp.logical_not(minp_out)
    filtered = jnp.where(final_mask, ps, 0.0)
    denom = jnp.sum(filtered, axis=-1, keepdims=True)
    denom_safe = jnp.where(denom == 0.0, 1.0, denom)
    normed = filtered / denom_safe
    normed = jnp.where((denom == 0.0) & (offs == 0), 1.0, normed)
    logp = jnp.where(normed > 0.0, jnp.log(jnp.maximum(normed, 1e-38)),
                     -jnp.inf)
    score = logp + g
    smax = jnp.max(score, axis=-1, keepdims=True)
    jstar = jnp.min(jnp.where(score == smax, offs, _K), axis=-1, keepdims=True)
    sel = offs == jstar
    tok = jnp.sum(jnp.where(sel, pidx, 0), axis=-1, keepdims=True)
    std_probs = jnp.where(sel, 1.0, 0.0).astype(jnp.float32)
    std_idx = jnp.where(sel, tok, 0)
    is_soft = sm > 0
    outp_ref[...] = jnp.where(is_soft, normed, std_probs)
    outi_ref[...] = jnp.where(is_soft, pidx, std_idx)
    tok_ref[...] = tok


@jax.jit
def _run(logits, top_p, top_k, min_p, soft_mask):
    top_k = jnp.maximum(top_k, 1)
    vals, ids = _sc_topk(logits.reshape(-1))
    vals = vals.reshape(_B, _K)
    ids = ids.reshape(_B, _K)
    m, z = pl.pallas_call(
        _mz_body,
        grid=(16,),
        in_specs=[pl.BlockSpec((8, _V), lambda i: (i, 0))],
        out_specs=(pl.BlockSpec((8, 1), lambda i: (i, 0)),
                   pl.BlockSpec((8, 1), lambda i: (i, 0))),
        out_shape=(jax.ShapeDtypeStruct((_B, 1), jnp.float32),
                   jax.ShapeDtypeStruct((_B, 1), jnp.float32)),
    )(logits)
    noise = jax.random.gumbel(jax.random.key(1234), (_B, _K), jnp.float32)

    outp, outi, tok = pl.pallas_call(
        _sample_body,
        out_shape=(
            jax.ShapeDtypeStruct((_B, _K), jnp.float32),
            jax.ShapeDtypeStruct((_B, _K), jnp.int32),
            jax.ShapeDtypeStruct((_B, 1), jnp.int32),
        ),
    )(vals, ids, m, z,
      top_p[:, None], top_k[:, None], min_p[:, None],
      soft_mask[:, None].astype(jnp.int32), noise)
    return outp, outi, tok[:, 0]


def kernel(logits, top_p, top_k, min_p, soft_mask):
    return _run(logits, top_p, top_k, min_p, soft_mask)
